# baseline (speedup 1.0000x reference)
import jax
import jax.numpy as jnp
from jax.experimental import pallas as pl
from jax.experimental.pallas import tpu as pltpu

T, S = 256, 16
HIDDEN = 1024
BATCH = 4
TH = 128
NBLK = BATCH * (T // TH)


def _body(t_ref, s_ref, g_ref, b_ref, x_ref, o_ref):
    pos = t_ref[0][:, None, :] + s_ref[...][None, :, :]
    mean = jnp.mean(pos, axis=-1, keepdims=True)
    cen = pos - mean
    var = jnp.mean(cen * cen, axis=-1, keepdims=True)
    pos = cen * jax.lax.rsqrt(var + 1e-6) * g_ref[0] + b_ref[0]
    o_ref[...] = x_ref[...] + pos[None]


def kernel(inputs, temporal_embedding, spectoral_embedding, ln_scale, ln_bias):
    x = inputs.reshape(NBLK, TH, S, HIDDEN)
    t_emb = temporal_embedding.reshape(T // TH, TH, HIDDEN)
    out = pl.pallas_call(
        _body,
        grid=(NBLK,),
        in_specs=[
            pl.BlockSpec((1, TH, HIDDEN), lambda i: (i % (T // TH), 0, 0)),
            pl.BlockSpec((S, HIDDEN), lambda i: (0, 0)),
            pl.BlockSpec((1, HIDDEN), lambda i: (0, 0)),
            pl.BlockSpec((1, HIDDEN), lambda i: (0, 0)),
            pl.BlockSpec((1, TH, S, HIDDEN), lambda i: (i, 0, 0, 0)),
        ],
        out_specs=pl.BlockSpec((1, TH, S, HIDDEN), lambda i: (i, 0, 0, 0)),
        out_shape=jax.ShapeDtypeStruct((NBLK, TH, S, HIDDEN), jnp.float32),
        compiler_params=pltpu.CompilerParams(
            dimension_semantics=("arbitrary",)),
    )(
        t_emb,
        spectoral_embedding,
        ln_scale.reshape(1, HIDDEN),
        ln_bias.reshape(1, HIDDEN),
        x,
    )
    return out.reshape(BATCH, 1, T * S, HIDDEN)


# speedup vs baseline: 1.0198x; 1.0198x over previous
import jax
import jax.numpy as jnp
from jax.experimental import pallas as pl
from jax.experimental.pallas import tpu as pltpu

T, S = 256, 16
HIDDEN = 1024
BATCH = 4
TH = 128
NBLK = BATCH * (T // TH)


def _body(t_ref, s_ref, g_ref, b_ref, x_ref, o_ref, pos_ref):
    i = pl.program_id(0)
    half = jax.lax.rem(i, T // TH)

    @pl.when(i < T // TH)
    def _compute_pos():
        pos = t_ref[0][:, None, :] + s_ref[...][None, :, :]
        mean = jnp.mean(pos, axis=-1, keepdims=True)
        cen = pos - mean
        var = jnp.mean(cen * cen, axis=-1, keepdims=True)
        pos = cen * jax.lax.rsqrt(var + 1e-6) * g_ref[0] + b_ref[0]
        pos_ref[pl.ds(half, 1)] = pos[None]

    o_ref[...] = x_ref[...] + pos_ref[pl.ds(half, 1)]


def kernel(inputs, temporal_embedding, spectoral_embedding, ln_scale, ln_bias):
    x = inputs.reshape(NBLK, TH, S, HIDDEN)
    t_emb = temporal_embedding.reshape(T // TH, TH, HIDDEN)
    out = pl.pallas_call(
        _body,
        grid=(NBLK,),
        in_specs=[
            pl.BlockSpec((1, TH, HIDDEN), lambda i: (i % (T // TH), 0, 0)),
            pl.BlockSpec((S, HIDDEN), lambda i: (0, 0)),
            pl.BlockSpec((1, HIDDEN), lambda i: (0, 0)),
            pl.BlockSpec((1, HIDDEN), lambda i: (0, 0)),
            pl.BlockSpec((1, TH, S, HIDDEN), lambda i: (i, 0, 0, 0)),
        ],
        out_specs=pl.BlockSpec((1, TH, S, HIDDEN), lambda i: (i, 0, 0, 0)),
        out_shape=jax.ShapeDtypeStruct((NBLK, TH, S, HIDDEN), jnp.float32),
        scratch_shapes=[pltpu.VMEM((T // TH, TH, S, HIDDEN), jnp.float32)],
        compiler_params=pltpu.CompilerParams(
            dimension_semantics=("arbitrary",)),
    )(
        t_emb,
        spectoral_embedding,
        ln_scale.reshape(1, HIDDEN),
        ln_bias.reshape(1, HIDDEN),
        x,
    )
    return out.reshape(BATCH, 1, T * S, HIDDEN)


# speedup vs baseline: 1.0242x; 1.0043x over previous
import jax
import jax.numpy as jnp
from jax.experimental import pallas as pl
from jax.experimental.pallas import tpu as pltpu

T, S = 256, 16
HIDDEN = 1024
BATCH = 4
TH = 128
NBLK = BATCH * (T // TH)


def _body(t_ref, s_ref, g_ref, b_ref, x_ref, o_ref, pos_ref):
    i = pl.program_id(0)
    half = i // BATCH

    @pl.when(jax.lax.rem(i, BATCH) == 0)
    def _compute_pos():
        pos = t_ref[0][:, None, :] + s_ref[...][None, :, :]
        mean = jnp.mean(pos, axis=-1, keepdims=True)
        cen = pos - mean
        var = jnp.mean(cen * cen, axis=-1, keepdims=True)
        pos = cen * jax.lax.rsqrt(var + 1e-6) * g_ref[0] + b_ref[0]
        pos_ref[pl.ds(half, 1)] = pos[None]

    o_ref[...] = x_ref[...] + pos_ref[pl.ds(half, 1)]


def kernel(inputs, temporal_embedding, spectoral_embedding, ln_scale, ln_bias):
    x = inputs.reshape(NBLK, TH, S, HIDDEN)
    t_emb = temporal_embedding.reshape(T // TH, TH, HIDDEN)
    out = pl.pallas_call(
        _body,
        grid=(NBLK,),
        in_specs=[
            pl.BlockSpec((1, TH, HIDDEN), lambda i: (i // BATCH, 0, 0)),
            pl.BlockSpec((S, HIDDEN), lambda i: (0, 0)),
            pl.BlockSpec((1, HIDDEN), lambda i: (0, 0)),
            pl.BlockSpec((1, HIDDEN), lambda i: (0, 0)),
            pl.BlockSpec((1, TH, S, HIDDEN),
                         lambda i: ((i % BATCH) * (T // TH) + i // BATCH, 0, 0, 0)),
        ],
        out_specs=pl.BlockSpec((1, TH, S, HIDDEN),
                               lambda i: ((i % BATCH) * (T // TH) + i // BATCH, 0, 0, 0)),
        out_shape=jax.ShapeDtypeStruct((NBLK, TH, S, HIDDEN), jnp.float32),
        scratch_shapes=[pltpu.VMEM((T // TH, TH, S, HIDDEN), jnp.float32)],
        compiler_params=pltpu.CompilerParams(
            dimension_semantics=("arbitrary",)),
    )(
        t_emb,
        spectoral_embedding,
        ln_scale.reshape(1, HIDDEN),
        ln_bias.reshape(1, HIDDEN),
        x,
    )
    return out.reshape(BATCH, 1, T * S, HIDDEN)


# speedup vs baseline: 1.0872x; 1.0615x over previous
import jax
import jax.numpy as jnp
from jax.experimental import pallas as pl
from jax.experimental.pallas import tpu as pltpu

T, S = 256, 16
HIDDEN = 1024
BATCH = 4
TT = 32


def _body(t_ref, s_ref, g_ref, b_ref, x_ref, o_ref):
    pos = t_ref[...][:, None, :] + s_ref[...][None, :, :]
    mean = jnp.mean(pos, axis=-1, keepdims=True)
    cen = pos - mean
    var = jnp.mean(cen * cen, axis=-1, keepdims=True)
    pos = cen * jax.lax.rsqrt(var + 1e-6) * g_ref[0] + b_ref[0]
    o_ref[...] = x_ref[...] + pos[None]


def kernel(inputs, temporal_embedding, spectoral_embedding, ln_scale, ln_bias):
    x = inputs.reshape(BATCH, T, S, HIDDEN)
    out = pl.pallas_call(
        _body,
        grid=(T // TT,),
        in_specs=[
            pl.BlockSpec((TT, HIDDEN), lambda i: (i, 0)),
            pl.BlockSpec((S, HIDDEN), lambda i: (0, 0)),
            pl.BlockSpec((1, HIDDEN), lambda i: (0, 0)),
            pl.BlockSpec((1, HIDDEN), lambda i: (0, 0)),
            pl.BlockSpec((BATCH, TT, S, HIDDEN), lambda i: (0, i, 0, 0)),
        ],
        out_specs=pl.BlockSpec((BATCH, TT, S, HIDDEN), lambda i: (0, i, 0, 0)),
        out_shape=jax.ShapeDtypeStruct((BATCH, T, S, HIDDEN), jnp.float32),
        compiler_params=pltpu.CompilerParams(
            dimension_semantics=("parallel",),
            vmem_limit_bytes=128 * 1024 * 1024),
    )(
        temporal_embedding,
        spectoral_embedding,
        ln_scale.reshape(1, HIDDEN),
        ln_bias.reshape(1, HIDDEN),
        x,
    )
    return out.reshape(BATCH, 1, T * S, HIDDEN)


# speedup vs baseline: 1.0884x; 1.0011x over previous
import jax
import jax.numpy as jnp
from jax.experimental import pallas as pl
from jax.experimental.pallas import tpu as pltpu

T, S = 256, 16
HIDDEN = 1024
BATCH = 4
TT = 32


def _body(t_ref, s_ref, g_ref, b_ref, x_ref, o_ref):
    t = t_ref[...]
    s = s_ref[...]
    inv_h = jnp.float32(1.0 / HIDDEN)
    sum_t = jnp.sum(t, axis=-1, keepdims=True)
    ssq_t = jnp.sum(t * t, axis=-1, keepdims=True)
    sum_s = jnp.sum(s, axis=-1)
    ssq_s = jnp.sum(s * s, axis=-1)
    cross = jax.lax.dot_general(
        t, s, (((1,), (1,)), ((), ())),
        preferred_element_type=jnp.float32)
    mu = (sum_t + sum_s[None, :]) * inv_h
    e2 = (ssq_t + ssq_s[None, :] + 2.0 * cross) * inv_h
    rstd = jax.lax.rsqrt(e2 - mu * mu + 1e-6)
    a = rstd[:, :, None]
    c = (mu * rstd)[:, :, None]
    pos = ((t[:, None, :] + s[None, :, :]) * a - c) * g_ref[0] + b_ref[0]
    o_ref[...] = x_ref[...] + pos[None]


def kernel(inputs, temporal_embedding, spectoral_embedding, ln_scale, ln_bias):
    x = inputs.reshape(BATCH, T, S, HIDDEN)
    out = pl.pallas_call(
        _body,
        grid=(T // TT,),
        in_specs=[
            pl.BlockSpec((TT, HIDDEN), lambda i: (i, 0)),
            pl.BlockSpec((S, HIDDEN), lambda i: (0, 0)),
            pl.BlockSpec((1, HIDDEN), lambda i: (0, 0)),
            pl.BlockSpec((1, HIDDEN), lambda i: (0, 0)),
            pl.BlockSpec((BATCH, TT, S, HIDDEN), lambda i: (0, i, 0, 0)),
        ],
        out_specs=pl.BlockSpec((BATCH, TT, S, HIDDEN), lambda i: (0, i, 0, 0)),
        out_shape=jax.ShapeDtypeStruct((BATCH, T, S, HIDDEN), jnp.float32),
        compiler_params=pltpu.CompilerParams(
            dimension_semantics=("parallel",),
            vmem_limit_bytes=128 * 1024 * 1024),
    )(
        temporal_embedding,
        spectoral_embedding,
        ln_scale.reshape(1, HIDDEN),
        ln_bias.reshape(1, HIDDEN),
        x,
    )
    return out.reshape(BATCH, 1, T * S, HIDDEN)
